# baseline (speedup 1.0000x reference)
import jax
import jax.numpy as jnp
from jax import lax
from jax.experimental import pallas as pl
from jax.experimental.pallas import tpu as pltpu
from jax.experimental.pallas import tpu_sc as plsc

N_USERS = 10000
N_NODES = 50000
E = 800000
NJ = 2
CHUNK_E = NJ * 128
NCH = 204
SUB_ROWS = NCH * NJ
ROWS128 = SUB_ROWS * 16
EP = ROWS128 * 128
ACC_CHUNK = 200
ACC_NCHUNK = N_NODES // ACC_CHUNK

_mesh = plsc.VectorSubcoreMesh(core_axis_name="c", subcore_axis_name="s")


def _segsum_body(feat, nbr2, tgt2, vals, out, acc,
                 tg0, tg1, tg2, tg3, tg4, tg5,
                 nb0, nb1, nb2, nb3, nb4, nb5,
                 vl0, vl1, vl2, vl3, vl4, vl5,
                 rows0, rows1, rows2,
                 si0, si1, si2, si3, si4, si5,
                 sg0, sg1, sg2, ss0, ss1, ss2, sz, sw):
    c = lax.axis_index("c")
    s = lax.axis_index("s")
    TG = [tg0, tg1, tg2, tg3, tg4, tg5]
    NB = [nb0, nb1, nb2, nb3, nb4, nb5]
    VL = [vl0, vl1, vl2, vl3, vl4, vl5]
    RW = [rows0, rows1, rows2]
    SI = [si0, si1, si2, si3, si4, si5]
    SG = [sg0, sg1, sg2]
    SS = [ss0, ss1, ss2]

    def zb(i, carry):
        rows0[i, pl.ds(0, 16)] = jnp.zeros((16,), jnp.float32)
        rows0[i, pl.ds(16, 16)] = jnp.zeros((16,), jnp.float32)
        return carry

    lax.fori_loop(0, ACC_CHUNK, zb, 0)

    def zcopy(cid):
        def f():
            pltpu.async_copy(rows0.at[pl.ds(0, ACC_CHUNK)],
                             acc.at[pl.ds(cid * ACC_CHUNK, ACC_CHUNK)], sz)
        return f

    def zwait():
        pltpu.make_async_copy(rows0.at[pl.ds(0, ACC_CHUNK)],
                              acc.at[pl.ds(0, ACC_CHUNK)], sz).wait()

    for k in range(16):
        cid = s + k * 16
        pl.when(cid < ACC_NCHUNK)(zcopy(cid))
    for k in range(16):
        cid = s + k * 16
        pl.when(cid < ACC_NCHUNK)(zwait)
    plsc.subcore_barrier()

    def idx_start(k, b):
        rb = s * SUB_ROWS + k * NJ
        pltpu.async_copy(tgt2.at[pl.ds(rb, NJ)], TG[b], SI[b])
        pltpu.async_copy(nbr2.at[pl.ds(c * ROWS128 + rb, NJ)], NB[b], SI[b])
        pltpu.async_copy(vals.at[pl.ds(rb * 128, CHUNK_E)], VL[b], SI[b])

    def idx_wait(b):
        pltpu.make_async_copy(tgt2.at[pl.ds(0, NJ)], TG[b], SI[b]).wait()
        pltpu.make_async_copy(nbr2.at[pl.ds(0, NJ)], NB[b], SI[b]).wait()
        pltpu.make_async_copy(vals.at[pl.ds(0, CHUNK_E)], VL[b], SI[b]).wait()

    def gather_start(b, r):
        for j in range(NJ):
            pltpu.async_copy(feat.at[NB[b].at[j]],
                             RW[r].at[pl.ds(j * 128, 128)], SG[r])

    def gather_wait(b, r):
        for j in range(NJ):
            pltpu.make_async_copy(feat.at[NB[b].at[j]],
                                  RW[r].at[pl.ds(j * 128, 128)], SG[r]).wait()

    def scatter_start(b, r):
        for j in range(NJ):
            pltpu.async_copy(RW[r].at[pl.ds(j * 128, 128)],
                             acc.at[TG[b].at[j]], SS[r], add=True)

    def scatter_wait(b, r):
        for j in range(NJ):
            pltpu.make_async_copy(RW[r].at[pl.ds(j * 128, 128)],
                                  acc.at[TG[b].at[j]], SS[r]).wait()

    def scale(b, r):
        def sc(g, carry):
            v16 = VL[b][pl.ds(g * 16, 16)]
            for u in range(16):
                e = g * 16 + u
                v = jnp.full((16,), v16[u], jnp.float32)
                RW[r][e, pl.ds(0, 16)] = RW[r][e, pl.ds(0, 16)] * v
                RW[r][e, pl.ds(16, 16)] = RW[r][e, pl.ds(16, 16)] * v
            return carry
        lax.fori_loop(0, CHUNK_E // 16, sc, 0)

    for j in range(4):
        idx_start(j, j)
    idx_wait(0)
    gather_start(0, 0)

    def hexa(g, carry):
        for b in range(6):
            k = 6 * g + b
            b1 = (b + 1) % 6
            b2 = (b + 4) % 6
            r = b % 3
            r1 = (b + 1) % 3
            pl.when(k + 1 < NCH)(lambda b1=b1: idx_wait(b1))
            if b < 2:
                pl.when(k >= 2)(lambda b2=b2, r1=r1: scatter_wait(b2, r1))
            else:
                scatter_wait(b2, r1)
            pl.when(k + 1 < NCH)(lambda b1=b1, r1=r1: gather_start(b1, r1))
            gather_wait(b, r)
            pl.when(k + 4 < NCH)(lambda k=k, b2=b2: idx_start(k + 4, b2))
            scale(b, r)
            scatter_start(b, r)
        return carry

    lax.fori_loop(0, NCH // 6, hexa, 0)
    scatter_wait((NCH - 2) % 6, (NCH - 2) % 3)
    scatter_wait((NCH - 1) % 6, (NCH - 1) % 3)
    plsc.subcore_barrier()

    def wcopy(cid):
        def f():
            pltpu.async_copy(
                acc.at[pl.ds(cid * ACC_CHUNK, ACC_CHUNK)],
                out.at[pl.ds(c * N_NODES + cid * ACC_CHUNK, ACC_CHUNK)], sw)
        return f

    def wwait():
        pltpu.make_async_copy(acc.at[pl.ds(0, ACC_CHUNK)],
                              out.at[pl.ds(0, ACC_CHUNK)], sw).wait()

    for k in range(16):
        cid = s + k * 16
        pl.when(cid < ACC_NCHUNK)(wcopy(cid))
    for k in range(16):
        cid = s + k * 16
        pl.when(cid < ACC_NCHUNK)(wwait)


_segsum = pl.kernel(
    _segsum_body,
    out_type=jax.ShapeDtypeStruct((2 * N_NODES, 32), jnp.float32),
    mesh=_mesh,
    scratch_types=(
        [pltpu.VMEM_SHARED((N_NODES, 32), jnp.float32)]
        + [pltpu.VMEM((NJ, 128), jnp.int32) for _ in range(12)]
        + [pltpu.VMEM((CHUNK_E,), jnp.float32) for _ in range(6)]
        + [pltpu.VMEM((CHUNK_E, 32), jnp.float32) for _ in range(3)]
        + [pltpu.SemaphoreType.DMA for _ in range(14)]
    ),
    compiler_params=pltpu.CompilerParams(needs_layout_passes=False, use_tc_tiling_on_sc=False),
)

R = 2000


def _dense0_body(f0, f1, h0, h1, w1, b1, w2, b2, out):
    f = jnp.concatenate([f0[...], f1[...]], axis=1)
    h = jnp.concatenate([h0[...], h1[...]], axis=1)
    sm = f + h
    pm = f * h
    z = (lax.dot_general(sm, w1[...], (((1,), (1,)), ((), ())),
                         preferred_element_type=jnp.float32)
         + lax.dot_general(pm, w2[...], (((1,), (1,)), ((), ())),
                           preferred_element_type=jnp.float32)
         + b1[...] + b2[...])
    y = jnp.where(z >= 0, z, 0.01 * z)
    n = jnp.sqrt(jnp.sum(y * y, axis=1, keepdims=True))
    y = y / jnp.maximum(n, 1e-12)
    out[0] = y[:, :32]
    out[1] = y[:, 32:]


_dense0 = pl.pallas_call(
    _dense0_body,
    grid=(N_NODES // R,),
    in_specs=[
        pl.BlockSpec((R, 32), lambda i: (i, 0)),
        pl.BlockSpec((R, 32), lambda i: (i + N_NODES // R, 0)),
        pl.BlockSpec((R, 32), lambda i: (i, 0)),
        pl.BlockSpec((R, 32), lambda i: (i + N_NODES // R, 0)),
        pl.BlockSpec((64, 64), lambda i: (0, 0)),
        pl.BlockSpec((1, 64), lambda i: (0, 0)),
        pl.BlockSpec((64, 64), lambda i: (0, 0)),
        pl.BlockSpec((1, 64), lambda i: (0, 0)),
    ],
    out_specs=pl.BlockSpec((2, R, 32), lambda i: (0, i, 0)),
    out_shape=jax.ShapeDtypeStruct((2, N_NODES, 32), jnp.float32),
)


def _dense1_body(f0, f1, h0, h1, w1, b1, w2, b2, out):
    f = jnp.concatenate([f0[...], f1[...]], axis=1)
    h = jnp.concatenate([h0[...], h1[...]], axis=1)
    sm = f + h
    pm = f * h
    z = (lax.dot_general(sm, w1[...], (((1,), (1,)), ((), ())),
                         preferred_element_type=jnp.float32)
         + lax.dot_general(pm, w2[...], (((1,), (1,)), ((), ())),
                           preferred_element_type=jnp.float32)
         + b1[...] + b2[...])
    y = jnp.where(z >= 0, z, 0.01 * z)
    n = jnp.sqrt(jnp.sum(y * y, axis=1, keepdims=True))
    y = y / jnp.maximum(n, 1e-12)
    out[...] = y


_dense1 = pl.pallas_call(
    _dense1_body,
    grid=(N_NODES // R,),
    in_specs=[
        pl.BlockSpec((R, 32), lambda i: (i, 0)),
        pl.BlockSpec((R, 32), lambda i: (i + N_NODES // R, 0)),
        pl.BlockSpec((R, 32), lambda i: (i, 0)),
        pl.BlockSpec((R, 32), lambda i: (i + N_NODES // R, 0)),
        pl.BlockSpec((32, 64), lambda i: (0, 0)),
        pl.BlockSpec((1, 32), lambda i: (0, 0)),
        pl.BlockSpec((32, 64), lambda i: (0, 0)),
        pl.BlockSpec((1, 32), lambda i: (0, 0)),
    ],
    out_specs=pl.BlockSpec((R, 32), lambda i: (i, 0)),
    out_shape=jax.ShapeDtypeStruct((N_NODES, 32), jnp.float32),
)


def _score_body(e0, e1, e2, uids, pids, nids, pos_out, neg_out,
                uv, uv2, pv, pv2, nv, nv2,
                u0a, u0b, u1a, u1b, u2t,
                p0a, p0b, p1a, p1b, p2t,
                n0a, n0b, n1a, n1b, n2t,
                posb, negb, sem):
    c = lax.axis_index("c")
    s = lax.axis_index("s")
    base = (s * 2 + c) * 128
    pltpu.sync_copy(uids.at[pl.ds(base, 128)], uv)
    pltpu.sync_copy(pids.at[pl.ds(base, 128)], pv)
    pltpu.sync_copy(nids.at[pl.ds(base, 128)], nv)

    def adj(i, carry):
        sl = pl.ds(i * 16, 16)
        uv2[sl] = uv[sl] + N_NODES
        pvv = pv[sl] + N_USERS
        pv[sl] = pvv
        pv2[sl] = pvv + N_NODES
        nvv = nv[sl] + N_USERS
        nv[sl] = nvv
        nv2[sl] = nvv + N_NODES
        return carry

    lax.fori_loop(0, 8, adj, 0)
    cps = []
    for (tab, iv, dst) in [
        (e0, uv, u0a), (e0, uv2, u0b), (e1, uv, u1a), (e1, uv2, u1b),
        (e2, uv, u2t),
        (e0, pv, p0a), (e0, pv2, p0b), (e1, pv, p1a), (e1, pv2, p1b),
        (e2, pv, p2t),
        (e0, nv, n0a), (e0, nv2, n0b), (e1, nv, n1a), (e1, nv2, n1b),
        (e2, nv, n2t),
    ]:
        cps.append(pltpu.async_copy(tab.at[iv], dst, sem))
    for cp in cps:
        cp.wait()
    iota16 = lax.iota(jnp.int32, 16)

    def dot(g, carry):
        row = g * 16 + iota16
        accp = jnp.zeros((16,), jnp.float32)
        accn = jnp.zeros((16,), jnp.float32)
        for (ut, pt, nt) in [(u0a, p0a, n0a), (u0b, p0b, n0b),
                             (u1a, p1a, n1a), (u1b, p1b, n1b),
                             (u2t, p2t, n2t)]:
            for cc in range(32):
                col = jnp.full((16,), cc, jnp.int32)
                a = plsc.load_gather(ut, [row, col])
                accp = accp + a * plsc.load_gather(pt, [row, col])
                accn = accn + a * plsc.load_gather(nt, [row, col])
        posb[pl.ds(g * 16, 16)] = accp
        negb[pl.ds(g * 16, 16)] = accn
        return carry

    lax.fori_loop(0, 8, dot, 0)
    pltpu.sync_copy(posb, pos_out.at[pl.ds(base, 128)])
    pltpu.sync_copy(negb, neg_out.at[pl.ds(base, 128)])


_score = pl.kernel(
    _score_body,
    out_type=(jax.ShapeDtypeStruct((4096,), jnp.float32),
              jax.ShapeDtypeStruct((4096,), jnp.float32)),
    mesh=_mesh,
    scratch_types=(
        [pltpu.VMEM((128,), jnp.int32) for _ in range(6)]
        + [pltpu.VMEM((128, 32), jnp.float32) for _ in range(15)]
        + [pltpu.VMEM((128,), jnp.float32) for _ in range(2)]
        + [pltpu.SemaphoreType.DMA]
    ),
    compiler_params=pltpu.CompilerParams(needs_layout_passes=False, use_tc_tiling_on_sc=False),
)


def kernel(adj_indices, adj_values, user_ids, pos_item_ids, neg_item_ids,
           user_embed, entity_embed,
           W1_0, b1_0, W2_0, b2_0, W1_1, b1_1, W2_1, b2_1):
    tgt = adj_indices[0].astype(jnp.int32)
    nbr = adj_indices[1].astype(jnp.int32)
    pad = EP - E
    zi = jnp.zeros((pad,), jnp.int32)
    tgt2 = jnp.concatenate([tgt, zi]).reshape(ROWS128, 128)
    nbr_p = jnp.concatenate([nbr, zi])
    nbr2 = jnp.concatenate([nbr_p, nbr_p + N_NODES]).reshape(2 * ROWS128, 128)
    vals = jnp.concatenate([adj_values.astype(jnp.float32),
                            jnp.zeros((pad,), jnp.float32)])
    e0 = jnp.concatenate([user_embed[:, :32], entity_embed[:, :32],
                          user_embed[:, 32:], entity_embed[:, 32:]], axis=0)
    h0 = _segsum(e0, nbr2, tgt2, vals)
    e1 = _dense0(e0, e0, h0, h0, W1_0, b1_0.reshape(1, 64),
                 W2_0, b2_0.reshape(1, 64)).reshape(2 * N_NODES, 32)
    h1 = _segsum(e1, nbr2, tgt2, vals)
    e2 = _dense1(e1, e1, h1, h1, W1_1, b1_1.reshape(1, 32),
                 W2_1, b2_1.reshape(1, 32))
    pos, neg = _score(e0, e1, e2,
                      user_ids.astype(jnp.int32),
                      pos_item_ids.astype(jnp.int32),
                      neg_item_ids.astype(jnp.int32))
    return pos, neg


# speedup vs baseline: 1.2998x; 1.2998x over previous
import jax
import jax.numpy as jnp
from jax import lax
from jax.experimental import pallas as pl
from jax.experimental.pallas import tpu as pltpu
from jax.experimental.pallas import tpu_sc as plsc

N_USERS = 10000
N_NODES = 50000
E = 800000
EP = 819200
ROWS128 = EP // 128
SUB_ROWS = ROWS128 // 16
NJ = 2
CHUNK_E = NJ * 128
NCH = SUB_ROWS // NJ
ACC_CHUNK = 200
ACC_NCHUNK = N_NODES // ACC_CHUNK

_mesh = plsc.VectorSubcoreMesh(core_axis_name="c", subcore_axis_name="s")


def _segsum_body(feat, nbr2, tgt2, vals, out, acc,
                 tg0, tg1, tg2, tg3, nb0, nb1, nb2, nb3,
                 vl0, vl1, vl2, vl3, rows0, rows1,
                 si0, si1, si2, si3, sg0, sg1, ss0, ss1, sz, sw):
    c = lax.axis_index("c")
    s = lax.axis_index("s")
    TG = [tg0, tg1, tg2, tg3]
    NB = [nb0, nb1, nb2, nb3]
    VL = [vl0, vl1, vl2, vl3]
    RW = [rows0, rows1]
    SI = [si0, si1, si2, si3]
    SG = [sg0, sg1]
    SS = [ss0, ss1]

    def zb(i, carry):
        rows0[i, pl.ds(0, 16)] = jnp.zeros((16,), jnp.float32)
        rows0[i, pl.ds(16, 16)] = jnp.zeros((16,), jnp.float32)
        return carry

    lax.fori_loop(0, ACC_CHUNK, zb, 0)

    def zcopy(cid):
        def f():
            pltpu.async_copy(rows0.at[pl.ds(0, ACC_CHUNK)],
                             acc.at[pl.ds(cid * ACC_CHUNK, ACC_CHUNK)], sz)
        return f

    def zwait():
        pltpu.make_async_copy(rows0.at[pl.ds(0, ACC_CHUNK)],
                              acc.at[pl.ds(0, ACC_CHUNK)], sz).wait()

    for k in range(16):
        cid = s + k * 16
        pl.when(cid < ACC_NCHUNK)(zcopy(cid))
    for k in range(16):
        cid = s + k * 16
        pl.when(cid < ACC_NCHUNK)(zwait)
    plsc.subcore_barrier()

    def idx_start(k, b):
        rb = s * SUB_ROWS + k * NJ
        pltpu.async_copy(tgt2.at[pl.ds(rb, NJ)], TG[b], SI[b])
        pltpu.async_copy(nbr2.at[pl.ds(c * ROWS128 + rb, NJ)], NB[b], SI[b])
        pltpu.async_copy(vals.at[pl.ds(rb * 128, CHUNK_E)], VL[b], SI[b])

    def idx_wait(b):
        pltpu.make_async_copy(tgt2.at[pl.ds(0, NJ)], TG[b], SI[b]).wait()
        pltpu.make_async_copy(nbr2.at[pl.ds(0, NJ)], NB[b], SI[b]).wait()
        pltpu.make_async_copy(vals.at[pl.ds(0, CHUNK_E)], VL[b], SI[b]).wait()

    def gather_start(b, r):
        for j in range(NJ):
            pltpu.async_copy(feat.at[NB[b].at[j]],
                             RW[r].at[pl.ds(j * 128, 128)], SG[r])

    def gather_wait(b, r):
        for j in range(NJ):
            pltpu.make_async_copy(feat.at[NB[b].at[j]],
                                  RW[r].at[pl.ds(j * 128, 128)], SG[r]).wait()

    def scatter_start(b, r):
        for j in range(NJ):
            pltpu.async_copy(RW[r].at[pl.ds(j * 128, 128)],
                             acc.at[TG[b].at[j]], SS[r], add=True)

    def scatter_wait(b, r):
        for j in range(NJ):
            pltpu.make_async_copy(RW[r].at[pl.ds(j * 128, 128)],
                                  acc.at[TG[b].at[j]], SS[r]).wait()

    def scale(b, r):
        def sc(g, carry):
            v16 = VL[b][pl.ds(g * 16, 16)]
            for u in range(16):
                e = g * 16 + u
                v = jnp.full((16,), v16[u], jnp.float32)
                RW[r][e, pl.ds(0, 16)] = RW[r][e, pl.ds(0, 16)] * v
                RW[r][e, pl.ds(16, 16)] = RW[r][e, pl.ds(16, 16)] * v
            return carry
        lax.fori_loop(0, CHUNK_E // 16, sc, 0)

    idx_start(0, 0)
    idx_start(1, 1)
    idx_start(2, 2)
    idx_wait(0)
    gather_start(0, 0)

    def quad(g, carry):
        for b in range(4):
            k = 4 * g + b
            b1 = (b + 1) % 4
            b3 = (b + 3) % 4
            r = b % 2
            r1 = (b + 1) % 2
            pl.when(k + 1 < NCH)(lambda b1=b1: idx_wait(b1))
            if b == 0:
                pl.when(g >= 1)(lambda b3=b3, r1=r1: scatter_wait(b3, r1))
            else:
                scatter_wait(b3, r1)
            pl.when(k + 1 < NCH)(lambda b1=b1, r1=r1: gather_start(b1, r1))
            gather_wait(b, r)
            pl.when(k + 3 < NCH)(lambda k=k, b3=b3: idx_start(k + 3, b3))
            scale(b, r)
            scatter_start(b, r)
        return carry

    lax.fori_loop(0, NCH // 4, quad, 0)
    scatter_wait(3, 1)
    plsc.subcore_barrier()

    def wcopy(cid):
        def f():
            pltpu.async_copy(
                acc.at[pl.ds(cid * ACC_CHUNK, ACC_CHUNK)],
                out.at[pl.ds(c * N_NODES + cid * ACC_CHUNK, ACC_CHUNK)], sw)
        return f

    def wwait():
        pltpu.make_async_copy(acc.at[pl.ds(0, ACC_CHUNK)],
                              out.at[pl.ds(0, ACC_CHUNK)], sw).wait()

    for k in range(16):
        cid = s + k * 16
        pl.when(cid < ACC_NCHUNK)(wcopy(cid))
    for k in range(16):
        cid = s + k * 16
        pl.when(cid < ACC_NCHUNK)(wwait)


_segsum = pl.kernel(
    _segsum_body,
    out_type=jax.ShapeDtypeStruct((2 * N_NODES, 32), jnp.float32),
    mesh=_mesh,
    scratch_types=(
        [pltpu.VMEM_SHARED((N_NODES, 32), jnp.float32)]
        + [pltpu.VMEM((NJ, 128), jnp.int32) for _ in range(8)]
        + [pltpu.VMEM((CHUNK_E,), jnp.float32) for _ in range(4)]
        + [pltpu.VMEM((CHUNK_E, 32), jnp.float32) for _ in range(2)]
        + [pltpu.SemaphoreType.DMA for _ in range(10)]
    ),
    compiler_params=pltpu.CompilerParams(needs_layout_passes=False, use_tc_tiling_on_sc=False),
)

R = 2000


def _dense0_body(f0, f1, h0, h1, w1, b1, w2, b2, out):
    f = jnp.concatenate([f0[...], f1[...]], axis=1)
    h = jnp.concatenate([h0[...], h1[...]], axis=1)
    sm = f + h
    pm = f * h
    z = (lax.dot_general(sm, w1[...], (((1,), (1,)), ((), ())),
                         preferred_element_type=jnp.float32)
         + lax.dot_general(pm, w2[...], (((1,), (1,)), ((), ())),
                           preferred_element_type=jnp.float32)
         + b1[...] + b2[...])
    y = jnp.where(z >= 0, z, 0.01 * z)
    n = jnp.sqrt(jnp.sum(y * y, axis=1, keepdims=True))
    y = y / jnp.maximum(n, 1e-12)
    out[0] = y[:, :32]
    out[1] = y[:, 32:]


_dense0 = pl.pallas_call(
    _dense0_body,
    grid=(N_NODES // R,),
    in_specs=[
        pl.BlockSpec((R, 32), lambda i: (i, 0)),
        pl.BlockSpec((R, 32), lambda i: (i + N_NODES // R, 0)),
        pl.BlockSpec((R, 32), lambda i: (i, 0)),
        pl.BlockSpec((R, 32), lambda i: (i + N_NODES // R, 0)),
        pl.BlockSpec((64, 64), lambda i: (0, 0)),
        pl.BlockSpec((1, 64), lambda i: (0, 0)),
        pl.BlockSpec((64, 64), lambda i: (0, 0)),
        pl.BlockSpec((1, 64), lambda i: (0, 0)),
    ],
    out_specs=pl.BlockSpec((2, R, 32), lambda i: (0, i, 0)),
    out_shape=jax.ShapeDtypeStruct((2, N_NODES, 32), jnp.float32),
)


def _dense1_body(f0, f1, h0, h1, w1, b1, w2, b2, out):
    f = jnp.concatenate([f0[...], f1[...]], axis=1)
    h = jnp.concatenate([h0[...], h1[...]], axis=1)
    sm = f + h
    pm = f * h
    z = (lax.dot_general(sm, w1[...], (((1,), (1,)), ((), ())),
                         preferred_element_type=jnp.float32)
         + lax.dot_general(pm, w2[...], (((1,), (1,)), ((), ())),
                           preferred_element_type=jnp.float32)
         + b1[...] + b2[...])
    y = jnp.where(z >= 0, z, 0.01 * z)
    n = jnp.sqrt(jnp.sum(y * y, axis=1, keepdims=True))
    y = y / jnp.maximum(n, 1e-12)
    out[...] = y


_dense1 = pl.pallas_call(
    _dense1_body,
    grid=(N_NODES // R,),
    in_specs=[
        pl.BlockSpec((R, 32), lambda i: (i, 0)),
        pl.BlockSpec((R, 32), lambda i: (i + N_NODES // R, 0)),
        pl.BlockSpec((R, 32), lambda i: (i, 0)),
        pl.BlockSpec((R, 32), lambda i: (i + N_NODES // R, 0)),
        pl.BlockSpec((32, 64), lambda i: (0, 0)),
        pl.BlockSpec((1, 32), lambda i: (0, 0)),
        pl.BlockSpec((32, 64), lambda i: (0, 0)),
        pl.BlockSpec((1, 32), lambda i: (0, 0)),
    ],
    out_specs=pl.BlockSpec((R, 32), lambda i: (i, 0)),
    out_shape=jax.ShapeDtypeStruct((N_NODES, 32), jnp.float32),
)


def _score_body(e0, e1, e2, uids, pids, nids, pos_out, neg_out,
                uv, uv2, pv, pv2, nv, nv2,
                u0a, u0b, u1a, u1b, u2t,
                p0a, p0b, p1a, p1b, p2t,
                n0a, n0b, n1a, n1b, n2t,
                posb, negb, sem):
    c = lax.axis_index("c")
    s = lax.axis_index("s")
    base = (s * 2 + c) * 128
    pltpu.sync_copy(uids.at[pl.ds(base, 128)], uv)
    pltpu.sync_copy(pids.at[pl.ds(base, 128)], pv)
    pltpu.sync_copy(nids.at[pl.ds(base, 128)], nv)

    def adj(i, carry):
        sl = pl.ds(i * 16, 16)
        uv2[sl] = uv[sl] + N_NODES
        pvv = pv[sl] + N_USERS
        pv[sl] = pvv
        pv2[sl] = pvv + N_NODES
        nvv = nv[sl] + N_USERS
        nv[sl] = nvv
        nv2[sl] = nvv + N_NODES
        return carry

    lax.fori_loop(0, 8, adj, 0)
    cps = []
    for (tab, iv, dst) in [
        (e0, uv, u0a), (e0, uv2, u0b), (e1, uv, u1a), (e1, uv2, u1b),
        (e2, uv, u2t),
        (e0, pv, p0a), (e0, pv2, p0b), (e1, pv, p1a), (e1, pv2, p1b),
        (e2, pv, p2t),
        (e0, nv, n0a), (e0, nv2, n0b), (e1, nv, n1a), (e1, nv2, n1b),
        (e2, nv, n2t),
    ]:
        cps.append(pltpu.async_copy(tab.at[iv], dst, sem))
    for cp in cps:
        cp.wait()
    iota16 = lax.iota(jnp.int32, 16)

    def dot(g, carry):
        row = g * 16 + iota16
        accp = jnp.zeros((16,), jnp.float32)
        accn = jnp.zeros((16,), jnp.float32)
        for (ut, pt, nt) in [(u0a, p0a, n0a), (u0b, p0b, n0b),
                             (u1a, p1a, n1a), (u1b, p1b, n1b),
                             (u2t, p2t, n2t)]:
            for cc in range(32):
                col = jnp.full((16,), cc, jnp.int32)
                a = plsc.load_gather(ut, [row, col])
                accp = accp + a * plsc.load_gather(pt, [row, col])
                accn = accn + a * plsc.load_gather(nt, [row, col])
        posb[pl.ds(g * 16, 16)] = accp
        negb[pl.ds(g * 16, 16)] = accn
        return carry

    lax.fori_loop(0, 8, dot, 0)
    pltpu.sync_copy(posb, pos_out.at[pl.ds(base, 128)])
    pltpu.sync_copy(negb, neg_out.at[pl.ds(base, 128)])


_score = pl.kernel(
    _score_body,
    out_type=(jax.ShapeDtypeStruct((4096,), jnp.float32),
              jax.ShapeDtypeStruct((4096,), jnp.float32)),
    mesh=_mesh,
    scratch_types=(
        [pltpu.VMEM((128,), jnp.int32) for _ in range(6)]
        + [pltpu.VMEM((128, 32), jnp.float32) for _ in range(15)]
        + [pltpu.VMEM((128,), jnp.float32) for _ in range(2)]
        + [pltpu.SemaphoreType.DMA]
    ),
    compiler_params=pltpu.CompilerParams(needs_layout_passes=False, use_tc_tiling_on_sc=False),
)


def kernel(adj_indices, adj_values, user_ids, pos_item_ids, neg_item_ids,
           user_embed, entity_embed,
           W1_0, b1_0, W2_0, b2_0, W1_1, b1_1, W2_1, b2_1):
    tgt = adj_indices[0].astype(jnp.int32)
    nbr = adj_indices[1].astype(jnp.int32)
    pad = EP - E
    zi = jnp.zeros((pad,), jnp.int32)
    tgt2 = jnp.concatenate([tgt, zi]).reshape(ROWS128, 128)
    nbr_p = jnp.concatenate([nbr, zi])
    nbr2 = jnp.concatenate([nbr_p, nbr_p + N_NODES]).reshape(2 * ROWS128, 128)
    vals = jnp.concatenate([adj_values.astype(jnp.float32),
                            jnp.zeros((pad,), jnp.float32)])
    e0 = jnp.concatenate([user_embed[:, :32], entity_embed[:, :32],
                          user_embed[:, 32:], entity_embed[:, 32:]], axis=0)
    h0 = _segsum(e0, nbr2, tgt2, vals)
    e1 = _dense0(e0, e0, h0, h0, W1_0, b1_0.reshape(1, 64),
                 W2_0, b2_0.reshape(1, 64)).reshape(2 * N_NODES, 32)
    h1 = _segsum(e1, nbr2, tgt2, vals)
    e2 = _dense1(e1, e1, h1, h1, W1_1, b1_1.reshape(1, 32),
                 W2_1, b2_1.reshape(1, 32))
    pos, neg = _score(e0, e1, e2,
                      user_ids.astype(jnp.int32),
                      pos_item_ids.astype(jnp.int32),
                      neg_item_ids.astype(jnp.int32))
    return pos, neg
